# baseline (speedup 1.0000x reference)
import jax
import jax.numpy as jnp
from jax.experimental import pallas as pl
from jax.experimental.pallas import tpu as pltpu

H = 32
W = 32
C = 256
B = 16
P = H * W
GRID = 50
NROWS = (H - 1) * GRID + W
NBLK = 1584


def _tc_body(w_ref, o_ref, buf4, sems):
    for i in range(H):
        chunk = w_ref[pl.ds(i * GRID, W), :]
        buf4[:, pl.ds(i, 1), :] = jnp.transpose(chunk, (1, 0))[:, None, :]
    copies = [
        pltpu.make_async_copy(buf4, o_ref.at[b], sems.at[b])
        for b in range(B)
    ]
    for cp in copies:
        cp.start()
    for cp in copies:
        cp.wait()


def kernel(mask, pos_embed_weight):
    bsz, h, w = mask.shape
    out = pl.pallas_call(
        _tc_body,
        grid=(1,),
        in_specs=[pl.BlockSpec((NBLK, C), lambda b: (0, 0))],
        out_specs=pl.BlockSpec(memory_space=pl.ANY),
        out_shape=jax.ShapeDtypeStruct((B, C, H, W), jnp.float32),
        scratch_shapes=[
            pltpu.VMEM((C, H, W), jnp.float32),
            pltpu.SemaphoreType.DMA((B,)),
        ],
    )(pos_embed_weight)
    return out


# speedup vs baseline: 1.0128x; 1.0128x over previous
import jax
import jax.numpy as jnp
from jax.experimental import pallas as pl
from jax.experimental.pallas import tpu as pltpu

H = 32
W = 32
C = 256
B = 16
P = H * W
GRID = 50
NBLK = 1584
BB = 4


def _tc_body(w_ref, o_ref, rows, t):
    @pl.when(pl.program_id(0) == 0)
    def _():
        for i in range(H):
            rows[pl.ds(i * W, W), :] = w_ref[pl.ds(i * GRID, W), :]
        t[...] = jnp.transpose(rows[...], (1, 0))

    tv = t[...].reshape(C, H, W)
    o_ref[...] = jnp.broadcast_to(tv[None], (BB, C, H, W))


def kernel(mask, pos_embed_weight):
    bsz, h, w = mask.shape
    out = pl.pallas_call(
        _tc_body,
        grid=(B // BB,),
        in_specs=[pl.BlockSpec((NBLK, C), lambda b: (0, 0))],
        out_specs=pl.BlockSpec((BB, C, H, W), lambda b: (b, 0, 0, 0)),
        out_shape=jax.ShapeDtypeStruct((B, C, H, W), jnp.float32),
        scratch_shapes=[
            pltpu.VMEM((P, C), jnp.float32),
            pltpu.VMEM((C, P), jnp.float32),
        ],
    )(pos_embed_weight)
    return out


# speedup vs baseline: 3.0249x; 2.9866x over previous
import jax
import jax.numpy as jnp
from jax.experimental import pallas as pl
from jax.experimental.pallas import tpu as pltpu

H = 32
W = 32
C = 256
B = 16
P = H * W
GRID = 50
NBLK = 1584


def _tc_body(w_ref, o_ref, rows, t8, sems):
    for i in range(H):
        rows[pl.ds(i * W, W), :] = w_ref[pl.ds(i * GRID, W), :]
    t = jnp.transpose(rows[...], (1, 0))
    t8[...] = t.reshape(C, 8, 128)
    copies = [
        pltpu.make_async_copy(t8, o_ref.at[b], sems.at[b]) for b in range(B)
    ]
    for cp in copies:
        cp.start()
    for cp in copies:
        cp.wait()


def kernel(mask, pos_embed_weight):
    bsz, h, w = mask.shape
    out = pl.pallas_call(
        _tc_body,
        grid=(1,),
        in_specs=[pl.BlockSpec((NBLK, C), lambda b: (0, 0))],
        out_specs=pl.BlockSpec(memory_space=pl.ANY),
        out_shape=jax.ShapeDtypeStruct((B, C, 8, 128), jnp.float32),
        scratch_shapes=[
            pltpu.VMEM((P, C), jnp.float32),
            pltpu.VMEM((C, 8, 128), jnp.float32),
            pltpu.SemaphoreType.DMA((B,)),
        ],
    )(pos_embed_weight)
    return out.reshape(bsz, C, h, w)


# speedup vs baseline: 6.2205x; 2.0564x over previous
import jax
import jax.numpy as jnp
from jax.experimental import pallas as pl
from jax.experimental.pallas import tpu as pltpu

H = 32
W = 32
C = 256
B = 16
P = H * W
GRID = 50
NBLK = 1584


def _tc_body(w_ref, o_ref, rows):
    for i in range(H):
        rows[pl.ds(i * W, W), :] = w_ref[pl.ds(i * GRID, W), :]
    o_ref[...] = jnp.transpose(rows[...], (1, 0)).reshape(C, 8, 128)


def kernel(mask, pos_embed_weight):
    bsz, h, w = mask.shape
    t8 = pl.pallas_call(
        _tc_body,
        grid=(1,),
        in_specs=[pl.BlockSpec((NBLK, C), lambda b: (0, 0))],
        out_specs=pl.BlockSpec((C, 8, 128), lambda b: (0, 0, 0)),
        out_shape=jax.ShapeDtypeStruct((C, 8, 128), jnp.float32),
        scratch_shapes=[pltpu.VMEM((P, C), jnp.float32)],
    )(pos_embed_weight)
    return jnp.broadcast_to(t8.reshape(1, C, h, w), (bsz, C, h, w))
